# baseline (speedup 1.0000x reference)
import functools

import jax
import jax.numpy as jnp
from jax import lax
from jax.experimental import pallas as pl
from jax.experimental.pallas import tpu as pltpu
from jax.experimental.pallas import tpu_sc as plsc

N_TRUE = 10000
E_TRUE = 320000
D = 128
D_HID = 256
D_OUT = 64

NC = 2
NS = 16
NW = NC * NS

CHUNK = 128
N_PAD = 10240
ROWS_PER_SUB = N_PAD // NS
K_CHUNKS = 80
EPT = K_CHUNKS * CHUNK
E_PAD = EPT * NW
NBUF = 2

BN_TC = 1024
N_BLOCKS = N_PAD // BN_TC


def _sc_segment_sum(table, src_p, dst_p, zeros2d):
  mesh = plsc.VectorSubcoreMesh(core_axis_name="c", subcore_axis_name="s")

  @functools.partial(
      pl.kernel,
      mesh=mesh,
      out_type=[
          jax.ShapeDtypeStruct((NC, N_PAD, D), jnp.float32),
          jax.ShapeDtypeStruct((NW, N_PAD), jnp.float32),
      ],
      compiler_params=pltpu.CompilerParams(needs_layout_passes=False),
      scratch_types=[
          pltpu.VMEM((K_CHUNKS, CHUNK), jnp.int32),
          pltpu.VMEM((K_CHUNKS, CHUNK), jnp.int32),
          pltpu.VMEM((CHUNK, D), jnp.float32),
          pltpu.VMEM((N_PAD,), jnp.float32),
          pltpu.VMEM_SHARED((N_PAD, D), jnp.float32),
          pltpu.SemaphoreType.DMA,
      ],
  )
  def seg_kernel(tab_hbm, src_hbm, dst_hbm, z_hbm, agg_out, deg_out,
                 src_all, dst_all, rows, deg_v, agg_sh, semg):
    c = lax.axis_index("c")
    s = lax.axis_index("s")
    wid = c * NS + s

    pltpu.sync_copy(src_hbm.at[wid], src_all)
    pltpu.sync_copy(dst_hbm.at[wid], dst_all)

    pltpu.sync_copy(z_hbm.at[pl.ds(s * ROWS_PER_SUB, ROWS_PER_SUB)],
                    agg_sh.at[pl.ds(s * ROWS_PER_SUB, ROWS_PER_SUB)])

    zeros16 = jnp.zeros((16,), jnp.float32)
    def _zero_deg(i, carry):
      deg_v[pl.ds(i * 16, 16)] = zeros16
      return carry
    lax.fori_loop(0, N_PAD // 16, _zero_deg, 0)

    plsc.subcore_barrier()

    ones16 = jnp.full((16,), 1.0, jnp.float32)

    def _edge_round(i, carry):
      pltpu.async_copy(tab_hbm.at[src_all.at[i]], rows, semg).wait()
      pltpu.sync_copy(rows, agg_sh.at[dst_all.at[i]], add=True)
      for j in range(CHUNK // 16):
        idx = dst_all[i, pl.ds(j * 16, 16)]
        plsc.addupdate_scatter(deg_v, [idx], ones16)
      return carry

    lax.fori_loop(0, K_CHUNKS, _edge_round, 0)

    plsc.subcore_barrier()

    pltpu.sync_copy(agg_sh.at[pl.ds(s * ROWS_PER_SUB, ROWS_PER_SUB)],
                    agg_out.at[c, pl.ds(s * ROWS_PER_SUB, ROWS_PER_SUB)])
    pltpu.sync_copy(deg_v, deg_out.at[wid])

  return seg_kernel(table, src_p, dst_p, zeros2d)


def _tc_decode_body(x_ref, agg_ref, deg_ref, we_ref, be_ref, wd1_ref, bd1_ref,
                    wd2_ref, bd2_ref, out_ref, u_acc):
  i = pl.program_id(0)

  @pl.when(i == 0)
  def _():
    u_acc[...] = jnp.zeros((8, D), jnp.float32)

  agg = agg_ref[0] + agg_ref[1]
  deg = jnp.sum(deg_ref[...], axis=0)[:, None]
  mean = agg / jnp.maximum(deg, 1.0)
  z = (x_ref[...] + mean) @ we_ref[...] + be_ref[...]
  h = jnp.maximum(z, 0.0)

  row = i * BN_TC + lax.broadcasted_iota(jnp.int32, (BN_TC, 1), 0)
  h = jnp.where(row < N_TRUE, h, 0.0)
  u_acc[0:1, :] += jnp.sum(h, axis=0, keepdims=True)

  @pl.when(i == N_BLOCKS - 1)
  def _():
    u = u_acc[0:1, :] * (1.0 / N_TRUE)
    hid = jnp.maximum(u @ wd1_ref[...] + bd1_ref[...], 0.0)
    out_ref[...] = hid @ wd2_ref[...] + bd2_ref[...]


def _tc_decode(x_pad, agg2, deg32, W_e, b_e, W_d1, b_d1, W_d2, b_d2):
  out = pl.pallas_call(
      _tc_decode_body,
      grid=(N_BLOCKS,),
      in_specs=[
          pl.BlockSpec((BN_TC, D), lambda i: (i, 0)),
          pl.BlockSpec((NC, BN_TC, D), lambda i: (0, i, 0)),
          pl.BlockSpec((NW, BN_TC), lambda i: (0, i)),
          pl.BlockSpec((D, D), lambda i: (0, 0)),
          pl.BlockSpec((1, D), lambda i: (0, 0)),
          pl.BlockSpec((D, D_HID), lambda i: (0, 0)),
          pl.BlockSpec((1, D_HID), lambda i: (0, 0)),
          pl.BlockSpec((D_HID, D_OUT), lambda i: (0, 0)),
          pl.BlockSpec((1, D_OUT), lambda i: (0, 0)),
      ],
      out_specs=pl.BlockSpec((1, D_OUT), lambda i: (0, 0)),
      out_shape=jax.ShapeDtypeStruct((1, D_OUT), jnp.float32),
      scratch_shapes=[pltpu.VMEM((8, D), jnp.float32)],
  )(x_pad, agg2, deg32, W_e, b_e.reshape(1, D), W_d1, b_d1.reshape(1, D_HID),
    W_d2, b_d2.reshape(1, D_OUT))
  return out.reshape(D_OUT)


@jax.jit
def kernel(x, edge_index, W_e, b_e, W_d1, b_d1, W_d2, b_d2):
  src = edge_index[0].astype(jnp.int32)
  dst = edge_index[1].astype(jnp.int32)
  pad_idx = jnp.full((E_PAD - E_TRUE,), N_TRUE, jnp.int32)
  src_p = jnp.concatenate([src, pad_idx]).reshape(NW, K_CHUNKS, CHUNK)
  dst_p = jnp.concatenate([dst, pad_idx]).reshape(NW, K_CHUNKS, CHUNK)
  x_pad = jnp.pad(x, ((0, N_PAD - N_TRUE), (0, 0)))
  zeros2d = jnp.zeros((N_PAD, D), jnp.float32)

  agg2, deg32 = _sc_segment_sum(x_pad, src_p, dst_p, zeros2d)
  return _tc_decode(x_pad, agg2, deg32, W_e, b_e, W_d1, b_d1, W_d2, b_d2)


# speedup vs baseline: 1.0191x; 1.0191x over previous
import functools

import jax
import jax.numpy as jnp
from jax import lax
from jax.experimental import pallas as pl
from jax.experimental.pallas import tpu as pltpu
from jax.experimental.pallas import tpu_sc as plsc

N_TRUE = 10000
E_TRUE = 320000
D = 128
D_HID = 256
D_OUT = 64

NC = 2
NS = 16
NW = NC * NS

CHUNK = 128
N_PAD = 10240
ROWS_PER_SUB = N_PAD // NS
K_CHUNKS = 80
EPT = K_CHUNKS * CHUNK
E_PAD = EPT * NW
NBUF = 4

BN_TC = 1024
N_BLOCKS = N_PAD // BN_TC


def _sc_segment_sum(table, src_p, dst_p, zeros2d):
  mesh = plsc.VectorSubcoreMesh(core_axis_name="c", subcore_axis_name="s")

  @functools.partial(
      pl.kernel,
      mesh=mesh,
      out_type=[
          jax.ShapeDtypeStruct((NC, N_PAD, D), jnp.float32),
          jax.ShapeDtypeStruct((NW, N_PAD), jnp.float32),
      ],
      compiler_params=pltpu.CompilerParams(needs_layout_passes=False),
      scratch_types=[
          [pltpu.VMEM((CHUNK,), jnp.int32)] * 2,
          [pltpu.VMEM((CHUNK,), jnp.int32)] * 2,
          pltpu.VMEM((CHUNK, D), jnp.float32),
          pltpu.VMEM((N_PAD,), jnp.float32),
          pltpu.VMEM_SHARED((N_PAD, D), jnp.float32),
          pltpu.SemaphoreType.DMA,
          [pltpu.SemaphoreType.DMA] * 2,
      ],
  )
  def seg_kernel(tab_hbm, src_hbm, dst_hbm, z_hbm, agg_out, deg_out,
                 src_v, dst_v, rows, deg_v, agg_sh, semg, semi):
    c = lax.axis_index("c")
    s = lax.axis_index("s")
    wid = c * NS + s

    pltpu.sync_copy(z_hbm.at[pl.ds(s * ROWS_PER_SUB, ROWS_PER_SUB)],
                    agg_sh.at[pl.ds(s * ROWS_PER_SUB, ROWS_PER_SUB)])

    zeros16 = jnp.zeros((16,), jnp.float32)
    def _zero_deg(i, carry):
      deg_v[pl.ds(i * 16, 16)] = zeros16
      return carry
    lax.fori_loop(0, N_PAD // 16, _zero_deg, 0)

    plsc.subcore_barrier()

    ones16 = jnp.full((16,), 1.0, jnp.float32)
    base = wid * EPT

    for b in range(2):
      off = base + b * CHUNK
      pltpu.async_copy(src_hbm.at[pl.ds(off, CHUNK)], src_v[b], semi[b])
      pltpu.async_copy(dst_hbm.at[pl.ds(off, CHUNK)], dst_v[b], semi[b])

    def _edge_round(i, carry):
      for b in range(2):
        k = 2 * i + b
        off = base + k * CHUNK
        pltpu.make_async_copy(src_hbm.at[pl.ds(off, CHUNK)], src_v[b],
                              semi[b]).wait()
        pltpu.make_async_copy(dst_hbm.at[pl.ds(off, CHUNK)], dst_v[b],
                              semi[b]).wait()
        gcopy = pltpu.async_copy(tab_hbm.at[src_v[b]], rows, semg)
        for j in range(CHUNK // 16):
          idx = dst_v[b][pl.ds(j * 16, 16)]
          plsc.addupdate_scatter(deg_v, [idx], ones16)
        gcopy.wait()
        pltpu.sync_copy(rows, agg_sh.at[dst_v[b]], add=True)
        offn = base + jnp.minimum(k + 2, K_CHUNKS - 1) * CHUNK
        pltpu.async_copy(src_hbm.at[pl.ds(offn, CHUNK)], src_v[b], semi[b])
        pltpu.async_copy(dst_hbm.at[pl.ds(offn, CHUNK)], dst_v[b], semi[b])
      return carry

    lax.fori_loop(0, K_CHUNKS // 2, _edge_round, 0)

    for b in range(2):
      off = base + (K_CHUNKS - 1) * CHUNK
      pltpu.make_async_copy(src_hbm.at[pl.ds(off, CHUNK)], src_v[b],
                            semi[b]).wait()
      pltpu.make_async_copy(dst_hbm.at[pl.ds(off, CHUNK)], dst_v[b],
                            semi[b]).wait()

    plsc.subcore_barrier()

    pltpu.sync_copy(agg_sh.at[pl.ds(s * ROWS_PER_SUB, ROWS_PER_SUB)],
                    agg_out.at[c, pl.ds(s * ROWS_PER_SUB, ROWS_PER_SUB)])
    pltpu.sync_copy(deg_v, deg_out.at[wid])

  return seg_kernel(table, src_p, dst_p, zeros2d)


def _tc_decode_body(x_ref, agg_ref, deg_ref, we_ref, be_ref, wd1_ref, bd1_ref,
                    wd2_ref, bd2_ref, out_ref, u_acc):
  i = pl.program_id(0)

  @pl.when(i == 0)
  def _():
    u_acc[...] = jnp.zeros((8, D), jnp.float32)

  agg = agg_ref[0] + agg_ref[1]
  deg = jnp.sum(deg_ref[...], axis=0)[:, None]
  mean = agg / jnp.maximum(deg, 1.0)
  z = (x_ref[...] + mean) @ we_ref[...] + be_ref[...]
  h = jnp.maximum(z, 0.0)

  row = i * BN_TC + lax.broadcasted_iota(jnp.int32, (BN_TC, 1), 0)
  h = jnp.where(row < N_TRUE, h, 0.0)
  u_acc[0:1, :] += jnp.sum(h, axis=0, keepdims=True)

  @pl.when(i == N_BLOCKS - 1)
  def _():
    u = u_acc[0:1, :] * (1.0 / N_TRUE)
    hid = jnp.maximum(u @ wd1_ref[...] + bd1_ref[...], 0.0)
    out_ref[...] = hid @ wd2_ref[...] + bd2_ref[...]


def _tc_decode(x_pad, agg2, deg32, W_e, b_e, W_d1, b_d1, W_d2, b_d2):
  out = pl.pallas_call(
      _tc_decode_body,
      grid=(N_BLOCKS,),
      in_specs=[
          pl.BlockSpec((BN_TC, D), lambda i: (i, 0)),
          pl.BlockSpec((NC, BN_TC, D), lambda i: (0, i, 0)),
          pl.BlockSpec((NW, BN_TC), lambda i: (0, i)),
          pl.BlockSpec((D, D), lambda i: (0, 0)),
          pl.BlockSpec((1, D), lambda i: (0, 0)),
          pl.BlockSpec((D, D_HID), lambda i: (0, 0)),
          pl.BlockSpec((1, D_HID), lambda i: (0, 0)),
          pl.BlockSpec((D_HID, D_OUT), lambda i: (0, 0)),
          pl.BlockSpec((1, D_OUT), lambda i: (0, 0)),
      ],
      out_specs=pl.BlockSpec((1, D_OUT), lambda i: (0, 0)),
      out_shape=jax.ShapeDtypeStruct((1, D_OUT), jnp.float32),
      scratch_shapes=[pltpu.VMEM((8, D), jnp.float32)],
  )(x_pad, agg2, deg32, W_e, b_e.reshape(1, D), W_d1, b_d1.reshape(1, D_HID),
    W_d2, b_d2.reshape(1, D_OUT))
  return out.reshape(D_OUT)


@jax.jit
def kernel(x, edge_index, W_e, b_e, W_d1, b_d1, W_d2, b_d2):
  src = edge_index[0].astype(jnp.int32)
  dst = edge_index[1].astype(jnp.int32)
  pad_idx = jnp.full((E_PAD - E_TRUE,), N_TRUE, jnp.int32)
  src_p = jnp.concatenate([src, pad_idx])
  dst_p = jnp.concatenate([dst, pad_idx])
  x_pad = jnp.pad(x, ((0, N_PAD - N_TRUE), (0, 0)))
  zeros2d = jnp.zeros((N_PAD, D), jnp.float32)

  agg2, deg32 = _sc_segment_sum(x_pad, src_p, dst_p, zeros2d)
  return _tc_decode(x_pad, agg2, deg32, W_e, b_e, W_d1, b_d1, W_d2, b_d2)


# speedup vs baseline: 1.8802x; 1.8450x over previous
import functools

import jax
import jax.numpy as jnp
from jax import lax
from jax.experimental import pallas as pl
from jax.experimental.pallas import tpu as pltpu
from jax.experimental.pallas import tpu_sc as plsc

N_TRUE = 10000
E_TRUE = 320000
D = 128
D_HID = 256
D_OUT = 64

NC = 2
NS = 16
NW = NC * NS

CHUNK = 128
N_PAD = 10240
ROWS_PER_SUB = N_PAD // NS
K0 = 114
K1 = 44
TOTAL_CHUNKS = NS * (K0 + K1)
E_PAD = TOTAL_CHUNKS * CHUNK

BN_TC = 1024
N_BLOCKS = N_PAD // BN_TC


def _sc_segment_sum(table, src_p, dst_p, zeros2d):
  mesh = plsc.VectorSubcoreMesh(core_axis_name="c", subcore_axis_name="s")

  @functools.partial(
      pl.kernel,
      mesh=mesh,
      out_type=[
          jax.ShapeDtypeStruct((NC, N_PAD, D), jnp.float32),
          jax.ShapeDtypeStruct((NW, N_PAD), jnp.float32),
      ],
      compiler_params=pltpu.CompilerParams(needs_layout_passes=False),
      scratch_types=[
          [pltpu.VMEM((CHUNK,), jnp.int32)] * 2,
          [pltpu.VMEM((CHUNK,), jnp.int32)] * 2,
          pltpu.VMEM((CHUNK, D), jnp.float32),
          pltpu.VMEM((N_PAD,), jnp.float32),
          pltpu.VMEM_SHARED((N_PAD, D), jnp.float32),
          pltpu.SemaphoreType.DMA,
          [pltpu.SemaphoreType.DMA] * 2,
      ],
  )
  def seg_kernel(tab_hbm, src_hbm, dst_hbm, z_hbm, agg_out, deg_out,
                 src_v, dst_v, rows, deg_v, agg_sh, semg, semi):
    c = lax.axis_index("c")
    s = lax.axis_index("s")
    wid = c * NS + s

    pltpu.sync_copy(z_hbm.at[pl.ds(s * ROWS_PER_SUB, ROWS_PER_SUB)],
                    agg_sh.at[pl.ds(s * ROWS_PER_SUB, ROWS_PER_SUB)])

    zeros16 = jnp.zeros((16,), jnp.float32)
    def _zero_deg(i, carry):
      deg_v[pl.ds(i * 16, 16)] = zeros16
      return carry
    lax.fori_loop(0, N_PAD // 16, _zero_deg, 0)

    plsc.subcore_barrier()

    ones16 = jnp.full((16,), 1.0, jnp.float32)
    nk = jnp.where(c == 0, K0, K1)
    base_chunk = jnp.where(c == 0, s * K0, NS * K0 + s * K1)
    base = base_chunk * CHUNK
    last_off = base + (nk - 1) * CHUNK

    for b in range(2):
      off = base + b * CHUNK
      pltpu.async_copy(src_hbm.at[pl.ds(off, CHUNK)], src_v[b], semi[b])
      pltpu.async_copy(dst_hbm.at[pl.ds(off, CHUNK)], dst_v[b], semi[b])

    def _edge_round(i, carry):
      for b in range(2):
        k = 2 * i + b
        off = base + k * CHUNK
        pltpu.make_async_copy(src_hbm.at[pl.ds(off, CHUNK)], src_v[b],
                              semi[b]).wait()
        pltpu.make_async_copy(dst_hbm.at[pl.ds(off, CHUNK)], dst_v[b],
                              semi[b]).wait()
        gcopy = pltpu.async_copy(tab_hbm.at[src_v[b]], rows, semg)
        for j in range(CHUNK // 16):
          idx = dst_v[b][pl.ds(j * 16, 16)]
          plsc.addupdate_scatter(deg_v, [idx], ones16)
        gcopy.wait()
        pltpu.sync_copy(rows, agg_sh.at[dst_v[b]], add=True)
        offn = base + jnp.minimum(k + 2, nk - 1) * CHUNK
        pltpu.async_copy(src_hbm.at[pl.ds(offn, CHUNK)], src_v[b], semi[b])
        pltpu.async_copy(dst_hbm.at[pl.ds(offn, CHUNK)], dst_v[b], semi[b])
      return carry

    lax.fori_loop(0, nk // 2, _edge_round, 0)

    for b in range(2):
      pltpu.make_async_copy(src_hbm.at[pl.ds(last_off, CHUNK)], src_v[b],
                            semi[b]).wait()
      pltpu.make_async_copy(dst_hbm.at[pl.ds(last_off, CHUNK)], dst_v[b],
                            semi[b]).wait()

    plsc.subcore_barrier()

    pltpu.sync_copy(agg_sh.at[pl.ds(s * ROWS_PER_SUB, ROWS_PER_SUB)],
                    agg_out.at[c, pl.ds(s * ROWS_PER_SUB, ROWS_PER_SUB)])
    pltpu.sync_copy(deg_v, deg_out.at[wid])

  return seg_kernel(table, src_p, dst_p, zeros2d)


def _tc_decode_body(x_ref, agg_ref, deg_ref, we_ref, be_ref, wd1_ref, bd1_ref,
                    wd2_ref, bd2_ref, out_ref, u_acc):
  i = pl.program_id(0)

  @pl.when(i == 0)
  def _():
    u_acc[...] = jnp.zeros((8, D), jnp.float32)

  agg = agg_ref[0] + agg_ref[1]
  deg = jnp.sum(deg_ref[...], axis=0)[:, None]
  mean = agg / jnp.maximum(deg, 1.0)
  z = (x_ref[...] + mean) @ we_ref[...] + be_ref[...]
  h = jnp.maximum(z, 0.0)

  row = i * BN_TC + lax.broadcasted_iota(jnp.int32, (BN_TC, 1), 0)
  h = jnp.where(row < N_TRUE, h, 0.0)
  u_acc[0:1, :] += jnp.sum(h, axis=0, keepdims=True)

  @pl.when(i == N_BLOCKS - 1)
  def _():
    u = u_acc[0:1, :] * (1.0 / N_TRUE)
    hid = jnp.maximum(u @ wd1_ref[...] + bd1_ref[...], 0.0)
    out_ref[...] = hid @ wd2_ref[...] + bd2_ref[...]


def _tc_decode(x_pad, agg2, deg32, W_e, b_e, W_d1, b_d1, W_d2, b_d2):
  out = pl.pallas_call(
      _tc_decode_body,
      grid=(N_BLOCKS,),
      in_specs=[
          pl.BlockSpec((BN_TC, D), lambda i: (i, 0)),
          pl.BlockSpec((NC, BN_TC, D), lambda i: (0, i, 0)),
          pl.BlockSpec((NW, BN_TC), lambda i: (0, i)),
          pl.BlockSpec((D, D), lambda i: (0, 0)),
          pl.BlockSpec((1, D), lambda i: (0, 0)),
          pl.BlockSpec((D, D_HID), lambda i: (0, 0)),
          pl.BlockSpec((1, D_HID), lambda i: (0, 0)),
          pl.BlockSpec((D_HID, D_OUT), lambda i: (0, 0)),
          pl.BlockSpec((1, D_OUT), lambda i: (0, 0)),
      ],
      out_specs=pl.BlockSpec((1, D_OUT), lambda i: (0, 0)),
      out_shape=jax.ShapeDtypeStruct((1, D_OUT), jnp.float32),
      scratch_shapes=[pltpu.VMEM((8, D), jnp.float32)],
  )(x_pad, agg2, deg32, W_e, b_e.reshape(1, D), W_d1, b_d1.reshape(1, D_HID),
    W_d2, b_d2.reshape(1, D_OUT))
  return out.reshape(D_OUT)


@jax.jit
def kernel(x, edge_index, W_e, b_e, W_d1, b_d1, W_d2, b_d2):
  src = edge_index[0].astype(jnp.int32)
  dst = edge_index[1].astype(jnp.int32)
  pad_idx = jnp.full((E_PAD - E_TRUE,), N_TRUE, jnp.int32)
  src_p = jnp.concatenate([src, pad_idx])
  dst_p = jnp.concatenate([dst, pad_idx])
  x_pad = jnp.pad(x, ((0, N_PAD - N_TRUE), (0, 0)))
  zeros2d = jnp.zeros((N_PAD, D), jnp.float32)

  agg2, deg32 = _sc_segment_sum(x_pad, src_p, dst_p, zeros2d)
  return _tc_decode(x_pad, agg2, deg32, W_e, b_e, W_d1, b_d1, W_d2, b_d2)


# speedup vs baseline: 2.0279x; 1.0785x over previous
import functools

import jax
import jax.numpy as jnp
from jax import lax
from jax.experimental import pallas as pl
from jax.experimental.pallas import tpu as pltpu
from jax.experimental.pallas import tpu_sc as plsc

N_TRUE = 10000
E_TRUE = 320000
D = 128
D_HID = 256
D_OUT = 64

NC = 2
NS = 16
NW = NC * NS

CHUNK = 128
N_PAD = 10240
ROWS_PER_SUB = N_PAD // NS
K0 = 114
K1 = 44
TOTAL_CHUNKS = NS * (K0 + K1)
E_PAD = TOTAL_CHUNKS * CHUNK

BN_TC = 1024
N_BLOCKS = N_PAD // BN_TC


def _sc_segment_sum(table, src_p, dst_p, zeros2d):
  mesh = plsc.VectorSubcoreMesh(core_axis_name="c", subcore_axis_name="s")

  @functools.partial(
      pl.kernel,
      mesh=mesh,
      out_type=[
          jax.ShapeDtypeStruct((NC, N_PAD, D), jnp.float32),
          jax.ShapeDtypeStruct((NW, N_PAD), jnp.float32),
      ],
      compiler_params=pltpu.CompilerParams(needs_layout_passes=False),
      scratch_types=[
          [pltpu.VMEM((CHUNK,), jnp.int32)] * 2,
          [pltpu.VMEM((CHUNK,), jnp.int32)] * 2,
          pltpu.VMEM((CHUNK, D), jnp.float32),
          pltpu.VMEM((N_PAD,), jnp.float32),
          pltpu.VMEM_SHARED((N_PAD, D), jnp.float32),
          pltpu.SemaphoreType.DMA,
          [pltpu.SemaphoreType.DMA] * 2,
      ],
  )
  def seg_kernel(tab_hbm, src_hbm, dst_hbm, z_hbm, agg_out, deg_out,
                 src_v, dst_v, rows, deg_v, agg_sh, semg, semi):
    c = lax.axis_index("c")
    s = lax.axis_index("s")
    wid = c * NS + s

    pltpu.sync_copy(z_hbm.at[pl.ds(s * ROWS_PER_SUB, ROWS_PER_SUB)],
                    agg_sh.at[pl.ds(s * ROWS_PER_SUB, ROWS_PER_SUB)])

    zeros16 = jnp.zeros((16,), jnp.float32)
    def _zero_deg(i, carry):
      deg_v[pl.ds(i * 16, 16)] = zeros16
      return carry
    lax.fori_loop(0, N_PAD // 16, _zero_deg, 0)

    plsc.subcore_barrier()

    ones16 = jnp.full((16,), 1.0, jnp.float32)
    nk = jnp.where(c == 0, K0, K1)
    base_chunk = jnp.where(c == 0, s * K0, NS * K0 + s * K1)
    base = base_chunk * CHUNK
    last_off = base + (nk - 1) * CHUNK

    for b in range(2):
      off = base + b * CHUNK
      pltpu.async_copy(src_hbm.at[pl.ds(off, CHUNK)], src_v[b], semi[b])
      pltpu.async_copy(dst_hbm.at[pl.ds(off, CHUNK)], dst_v[b], semi[b])

    def _edge_round(i, carry):
      for b in range(2):
        k = 2 * i + b
        off = base + k * CHUNK
        pltpu.make_async_copy(src_hbm.at[pl.ds(off, CHUNK)], src_v[b],
                              semi[b]).wait()
        pltpu.make_async_copy(dst_hbm.at[pl.ds(off, CHUNK)], dst_v[b],
                              semi[b]).wait()
        gcopy = pltpu.async_copy(tab_hbm.at[src_v[b]], rows, semg)
        for j in range(CHUNK // 16):
          idx = dst_v[b][pl.ds(j * 16, 16)]
          plsc.addupdate_scatter(deg_v, [idx], ones16)
        gcopy.wait()
        pltpu.sync_copy(rows, agg_sh.at[dst_v[b]], add=True)
        offn = base + jnp.minimum(k + 2, nk - 1) * CHUNK
        pltpu.async_copy(src_hbm.at[pl.ds(offn, CHUNK)], src_v[b], semi[b])
        pltpu.async_copy(dst_hbm.at[pl.ds(offn, CHUNK)], dst_v[b], semi[b])
      return carry

    lax.fori_loop(0, nk // 2, _edge_round, 0)

    for b in range(2):
      pltpu.make_async_copy(src_hbm.at[pl.ds(last_off, CHUNK)], src_v[b],
                            semi[b]).wait()
      pltpu.make_async_copy(dst_hbm.at[pl.ds(last_off, CHUNK)], dst_v[b],
                            semi[b]).wait()

    plsc.subcore_barrier()

    pltpu.sync_copy(agg_sh.at[pl.ds(s * ROWS_PER_SUB, ROWS_PER_SUB)],
                    agg_out.at[c, pl.ds(s * ROWS_PER_SUB, ROWS_PER_SUB)])
    pltpu.sync_copy(deg_v, deg_out.at[wid])

  return seg_kernel(table, src_p, dst_p, zeros2d)


def _tc_decode_body(x_ref, agg_ref, deg_ref, we_ref, be_ref, wd1_ref, bd1_ref,
                    wd2_ref, bd2_ref, out_ref, u_acc):
  i = pl.program_id(0)

  @pl.when(i == 0)
  def _():
    u_acc[...] = jnp.zeros((8, D), jnp.float32)

  agg = agg_ref[0] + agg_ref[1]
  deg = jnp.sum(deg_ref[...], axis=0)[:, None]
  mean = agg / jnp.maximum(deg, 1.0)
  z = (x_ref[...] + mean) @ we_ref[...] + be_ref[...]
  h = jnp.maximum(z, 0.0)

  row = i * BN_TC + lax.broadcasted_iota(jnp.int32, (BN_TC, 1), 0)
  h = jnp.where(row < N_TRUE, h, 0.0)
  u_acc[0:1, :] += jnp.sum(h, axis=0, keepdims=True)

  @pl.when(i == N_BLOCKS - 1)
  def _():
    u = u_acc[0:1, :] * (1.0 / N_TRUE)
    hid = jnp.maximum(u @ wd1_ref[...] + bd1_ref[...], 0.0)
    out_ref[...] = hid @ wd2_ref[...] + bd2_ref[...]


def _tc_decode(x_pad, agg2, deg32, W_e, b_e, W_d1, b_d1, W_d2, b_d2):
  out = pl.pallas_call(
      _tc_decode_body,
      grid=(N_BLOCKS,),
      in_specs=[
          pl.BlockSpec((BN_TC, D), lambda i: (i, 0)),
          pl.BlockSpec((NC, BN_TC, D), lambda i: (0, i, 0)),
          pl.BlockSpec((NW, BN_TC), lambda i: (0, i)),
          pl.BlockSpec((D, D), lambda i: (0, 0)),
          pl.BlockSpec((1, D), lambda i: (0, 0)),
          pl.BlockSpec((D, D_HID), lambda i: (0, 0)),
          pl.BlockSpec((1, D_HID), lambda i: (0, 0)),
          pl.BlockSpec((D_HID, D_OUT), lambda i: (0, 0)),
          pl.BlockSpec((1, D_OUT), lambda i: (0, 0)),
      ],
      out_specs=pl.BlockSpec((1, D_OUT), lambda i: (0, 0)),
      out_shape=jax.ShapeDtypeStruct((1, D_OUT), jnp.float32),
      scratch_shapes=[pltpu.VMEM((8, D), jnp.float32)],
  )(x_pad, agg2, deg32, W_e, b_e.reshape(1, D), W_d1, b_d1.reshape(1, D_HID),
    W_d2, b_d2.reshape(1, D_OUT))
  return out.reshape(D_OUT)


@jax.jit
def kernel(x, edge_index, W_e, b_e, W_d1, b_d1, W_d2, b_d2):
  src = edge_index[0].astype(jnp.int32)
  dst = edge_index[1].astype(jnp.int32)
  pad_idx = N_TRUE + jnp.arange(E_PAD - E_TRUE, dtype=jnp.int32) % (
      N_PAD - N_TRUE)
  src_p = jnp.concatenate([src, pad_idx])
  dst_p = jnp.concatenate([dst, pad_idx])
  x_pad = jnp.pad(x, ((0, N_PAD - N_TRUE), (0, 0)))
  zeros2d = jnp.zeros((N_PAD, D), jnp.float32)

  agg2, deg32 = _sc_segment_sum(x_pad, src_p, dst_p, zeros2d)
  return _tc_decode(x_pad, agg2, deg32, W_e, b_e, W_d1, b_d1, W_d2, b_d2)


# speedup vs baseline: 2.4972x; 1.2314x over previous
import functools

import jax
import jax.numpy as jnp
from jax import lax
from jax.experimental import pallas as pl
from jax.experimental.pallas import tpu as pltpu
from jax.experimental.pallas import tpu_sc as plsc

N_TRUE = 10000
E_TRUE = 320000
D = 128
D_HID = 256
D_OUT = 64

NC = 2
NS = 16
NW = NC * NS

CHUNK = 128
N_PAD = 10240
ROWS_PER_SUB = N_PAD // NS
K0 = 86
K1 = 72
TOTAL_CHUNKS = NS * (K0 + K1)
E_PAD = TOTAL_CHUNKS * CHUNK

BN_TC = 1024
N_BLOCKS = N_PAD // BN_TC


def _sc_segment_sum(table, src_p, dst_p, zeros2d):
  mesh = plsc.VectorSubcoreMesh(core_axis_name="c", subcore_axis_name="s")

  @functools.partial(
      pl.kernel,
      mesh=mesh,
      out_type=[
          jax.ShapeDtypeStruct((NC, N_PAD, D), jnp.float32),
          jax.ShapeDtypeStruct((NW, N_PAD), jnp.float32),
      ],
      compiler_params=pltpu.CompilerParams(needs_layout_passes=False),
      scratch_types=[
          [pltpu.VMEM((CHUNK,), jnp.int32)] * 2,
          [pltpu.VMEM((CHUNK,), jnp.int32)] * 2,
          pltpu.VMEM((CHUNK, D), jnp.float32),
          pltpu.VMEM((N_PAD,), jnp.float32),
          pltpu.VMEM_SHARED((N_PAD, D), jnp.float32),
          pltpu.SemaphoreType.DMA,
          [pltpu.SemaphoreType.DMA] * 2,
      ],
  )
  def seg_kernel(tab_hbm, src_hbm, dst_hbm, z_hbm, agg_out, deg_out,
                 src_v, dst_v, rows, deg_v, agg_sh, semg, semi):
    c = lax.axis_index("c")
    s = lax.axis_index("s")
    wid = c * NS + s

    pltpu.sync_copy(z_hbm.at[pl.ds(s * ROWS_PER_SUB, ROWS_PER_SUB)],
                    agg_sh.at[pl.ds(s * ROWS_PER_SUB, ROWS_PER_SUB)])

    zeros16 = jnp.zeros((16,), jnp.float32)
    def _zero_deg(i, carry):
      deg_v[pl.ds(i * 16, 16)] = zeros16
      return carry
    lax.fori_loop(0, N_PAD // 16, _zero_deg, 0)

    plsc.subcore_barrier()

    ones16 = jnp.full((16,), 1.0, jnp.float32)
    nk = jnp.where(c == 0, K0, K1)
    base_chunk = jnp.where(c == 0, s * K0, NS * K0 + s * K1)
    base = base_chunk * CHUNK
    last_off = base + (nk - 1) * CHUNK

    for b in range(2):
      off = base + b * CHUNK
      pltpu.async_copy(src_hbm.at[pl.ds(off, CHUNK)], src_v[b], semi[b])
      pltpu.async_copy(dst_hbm.at[pl.ds(off, CHUNK)], dst_v[b], semi[b])

    def _edge_round(i, carry):
      for b in range(2):
        k = 2 * i + b
        off = base + k * CHUNK
        pltpu.make_async_copy(src_hbm.at[pl.ds(off, CHUNK)], src_v[b],
                              semi[b]).wait()
        pltpu.make_async_copy(dst_hbm.at[pl.ds(off, CHUNK)], dst_v[b],
                              semi[b]).wait()
        gcopy = pltpu.async_copy(tab_hbm.at[src_v[b]], rows, semg)
        for j in range(CHUNK // 16):
          idx = dst_v[b][pl.ds(j * 16, 16)]
          plsc.addupdate_scatter(deg_v, [idx], ones16)
        gcopy.wait()
        pltpu.sync_copy(rows, agg_sh.at[dst_v[b]], add=True)
        offn = base + jnp.minimum(k + 2, nk - 1) * CHUNK
        pltpu.async_copy(src_hbm.at[pl.ds(offn, CHUNK)], src_v[b], semi[b])
        pltpu.async_copy(dst_hbm.at[pl.ds(offn, CHUNK)], dst_v[b], semi[b])
      return carry

    lax.fori_loop(0, nk // 2, _edge_round, 0)

    for b in range(2):
      pltpu.make_async_copy(src_hbm.at[pl.ds(last_off, CHUNK)], src_v[b],
                            semi[b]).wait()
      pltpu.make_async_copy(dst_hbm.at[pl.ds(last_off, CHUNK)], dst_v[b],
                            semi[b]).wait()

    plsc.subcore_barrier()

    pltpu.sync_copy(agg_sh.at[pl.ds(s * ROWS_PER_SUB, ROWS_PER_SUB)],
                    agg_out.at[c, pl.ds(s * ROWS_PER_SUB, ROWS_PER_SUB)])
    pltpu.sync_copy(deg_v, deg_out.at[wid])

  return seg_kernel(table, src_p, dst_p, zeros2d)


def _tc_decode_body(x_ref, agg_ref, deg_ref, we_ref, be_ref, wd1_ref, bd1_ref,
                    wd2_ref, bd2_ref, out_ref, u_acc):
  i = pl.program_id(0)

  @pl.when(i == 0)
  def _():
    u_acc[...] = jnp.zeros((8, D), jnp.float32)

  agg = agg_ref[0] + agg_ref[1]
  deg = jnp.sum(deg_ref[...], axis=0)[:, None]
  mean = agg / jnp.maximum(deg, 1.0)
  z = (x_ref[...] + mean) @ we_ref[...] + be_ref[...]
  h = jnp.maximum(z, 0.0)

  row = i * BN_TC + lax.broadcasted_iota(jnp.int32, (BN_TC, 1), 0)
  h = jnp.where(row < N_TRUE, h, 0.0)
  u_acc[0:1, :] += jnp.sum(h, axis=0, keepdims=True)

  @pl.when(i == N_BLOCKS - 1)
  def _():
    u = u_acc[0:1, :] * (1.0 / N_TRUE)
    hid = jnp.maximum(u @ wd1_ref[...] + bd1_ref[...], 0.0)
    out_ref[...] = hid @ wd2_ref[...] + bd2_ref[...]


def _tc_decode(x_pad, agg2, deg32, W_e, b_e, W_d1, b_d1, W_d2, b_d2):
  out = pl.pallas_call(
      _tc_decode_body,
      grid=(N_BLOCKS,),
      in_specs=[
          pl.BlockSpec((BN_TC, D), lambda i: (i, 0)),
          pl.BlockSpec((NC, BN_TC, D), lambda i: (0, i, 0)),
          pl.BlockSpec((NW, BN_TC), lambda i: (0, i)),
          pl.BlockSpec((D, D), lambda i: (0, 0)),
          pl.BlockSpec((1, D), lambda i: (0, 0)),
          pl.BlockSpec((D, D_HID), lambda i: (0, 0)),
          pl.BlockSpec((1, D_HID), lambda i: (0, 0)),
          pl.BlockSpec((D_HID, D_OUT), lambda i: (0, 0)),
          pl.BlockSpec((1, D_OUT), lambda i: (0, 0)),
      ],
      out_specs=pl.BlockSpec((1, D_OUT), lambda i: (0, 0)),
      out_shape=jax.ShapeDtypeStruct((1, D_OUT), jnp.float32),
      scratch_shapes=[pltpu.VMEM((8, D), jnp.float32)],
  )(x_pad, agg2, deg32, W_e, b_e.reshape(1, D), W_d1, b_d1.reshape(1, D_HID),
    W_d2, b_d2.reshape(1, D_OUT))
  return out.reshape(D_OUT)


@jax.jit
def kernel(x, edge_index, W_e, b_e, W_d1, b_d1, W_d2, b_d2):
  src = edge_index[0].astype(jnp.int32)
  dst = edge_index[1].astype(jnp.int32)
  pad_idx = N_TRUE + jnp.arange(E_PAD - E_TRUE, dtype=jnp.int32) % (
      N_PAD - N_TRUE)
  src_p = jnp.concatenate([src, pad_idx])
  dst_p = jnp.concatenate([dst, pad_idx])
  x_pad = jnp.pad(x, ((0, N_PAD - N_TRUE), (0, 0)))
  zeros2d = jnp.zeros((N_PAD, D), jnp.float32)

  agg2, deg32 = _sc_segment_sum(x_pad, src_p, dst_p, zeros2d)
  return _tc_decode(x_pad, agg2, deg32, W_e, b_e, W_d1, b_d1, W_d2, b_d2)


# speedup vs baseline: 2.6227x; 1.0503x over previous
import functools

import jax
import jax.numpy as jnp
from jax import lax
from jax.experimental import pallas as pl
from jax.experimental.pallas import tpu as pltpu
from jax.experimental.pallas import tpu_sc as plsc

N_TRUE = 10000
E_TRUE = 320000
D = 128
D_HID = 256
D_OUT = 64

NC = 2
NS = 16
NW = NC * NS

CHUNK = 128
N_PAD = 10240
ROWS_PER_SUB = N_PAD // NS
K0 = 80
K1 = 78
TOTAL_CHUNKS = NS * (K0 + K1)
E_PAD = TOTAL_CHUNKS * CHUNK

BN_TC = 1024
N_BLOCKS = N_PAD // BN_TC


def _sc_segment_sum(table, src_p, dst_p, zeros2d):
  mesh = plsc.VectorSubcoreMesh(core_axis_name="c", subcore_axis_name="s")

  @functools.partial(
      pl.kernel,
      mesh=mesh,
      out_type=[
          jax.ShapeDtypeStruct((NC, N_PAD, D), jnp.float32),
          jax.ShapeDtypeStruct((NW, N_PAD), jnp.float32),
      ],
      compiler_params=pltpu.CompilerParams(needs_layout_passes=False),
      scratch_types=[
          [pltpu.VMEM((CHUNK,), jnp.int32)] * 2,
          [pltpu.VMEM((CHUNK,), jnp.int32)] * 2,
          pltpu.VMEM((CHUNK, D), jnp.float32),
          pltpu.VMEM((N_PAD,), jnp.float32),
          pltpu.VMEM_SHARED((N_PAD, D), jnp.float32),
          pltpu.SemaphoreType.DMA,
          [pltpu.SemaphoreType.DMA] * 2,
      ],
  )
  def seg_kernel(tab_hbm, src_hbm, dst_hbm, z_hbm, agg_out, deg_out,
                 src_v, dst_v, rows, deg_v, agg_sh, semg, semi):
    c = lax.axis_index("c")
    s = lax.axis_index("s")
    wid = c * NS + s

    pltpu.sync_copy(z_hbm.at[pl.ds(s * ROWS_PER_SUB, ROWS_PER_SUB)],
                    agg_sh.at[pl.ds(s * ROWS_PER_SUB, ROWS_PER_SUB)])

    zeros16 = jnp.zeros((16,), jnp.float32)
    def _zero_deg(i, carry):
      deg_v[pl.ds(i * 16, 16)] = zeros16
      return carry
    lax.fori_loop(0, N_PAD // 16, _zero_deg, 0)

    plsc.subcore_barrier()

    ones16 = jnp.full((16,), 1.0, jnp.float32)
    nk = jnp.where(c == 0, K0, K1)
    base_chunk = jnp.where(c == 0, s * K0, NS * K0 + s * K1)
    base = base_chunk * CHUNK
    last_off = base + (nk - 1) * CHUNK

    for b in range(2):
      off = base + b * CHUNK
      pltpu.async_copy(src_hbm.at[pl.ds(off, CHUNK)], src_v[b], semi[b])
      pltpu.async_copy(dst_hbm.at[pl.ds(off, CHUNK)], dst_v[b], semi[b])

    def _edge_round(i, carry):
      for b in range(2):
        k = 2 * i + b
        off = base + k * CHUNK
        pltpu.make_async_copy(src_hbm.at[pl.ds(off, CHUNK)], src_v[b],
                              semi[b]).wait()
        pltpu.make_async_copy(dst_hbm.at[pl.ds(off, CHUNK)], dst_v[b],
                              semi[b]).wait()
        gcopy = pltpu.async_copy(tab_hbm.at[src_v[b]], rows, semg)
        for j in range(CHUNK // 16):
          idx = dst_v[b][pl.ds(j * 16, 16)]
          plsc.addupdate_scatter(deg_v, [idx], ones16)
        gcopy.wait()
        pltpu.sync_copy(rows, agg_sh.at[dst_v[b]], add=True)
        offn = base + jnp.minimum(k + 2, nk - 1) * CHUNK
        pltpu.async_copy(src_hbm.at[pl.ds(offn, CHUNK)], src_v[b], semi[b])
        pltpu.async_copy(dst_hbm.at[pl.ds(offn, CHUNK)], dst_v[b], semi[b])
      return carry

    lax.fori_loop(0, nk // 2, _edge_round, 0)

    for b in range(2):
      pltpu.make_async_copy(src_hbm.at[pl.ds(last_off, CHUNK)], src_v[b],
                            semi[b]).wait()
      pltpu.make_async_copy(dst_hbm.at[pl.ds(last_off, CHUNK)], dst_v[b],
                            semi[b]).wait()

    plsc.subcore_barrier()

    pltpu.sync_copy(agg_sh.at[pl.ds(s * ROWS_PER_SUB, ROWS_PER_SUB)],
                    agg_out.at[c, pl.ds(s * ROWS_PER_SUB, ROWS_PER_SUB)])
    pltpu.sync_copy(deg_v, deg_out.at[wid])

  return seg_kernel(table, src_p, dst_p, zeros2d)


def _tc_decode_body(x_ref, agg_ref, deg_ref, we_ref, be_ref, wd1_ref, bd1_ref,
                    wd2_ref, bd2_ref, out_ref, u_acc):
  i = pl.program_id(0)

  @pl.when(i == 0)
  def _():
    u_acc[...] = jnp.zeros((8, D), jnp.float32)

  agg = agg_ref[0] + agg_ref[1]
  deg = jnp.sum(deg_ref[...], axis=0)[:, None]
  mean = agg / jnp.maximum(deg, 1.0)
  z = (x_ref[...] + mean) @ we_ref[...] + be_ref[...]
  h = jnp.maximum(z, 0.0)

  row = i * BN_TC + lax.broadcasted_iota(jnp.int32, (BN_TC, 1), 0)
  h = jnp.where(row < N_TRUE, h, 0.0)
  u_acc[0:1, :] += jnp.sum(h, axis=0, keepdims=True)

  @pl.when(i == N_BLOCKS - 1)
  def _():
    u = u_acc[0:1, :] * (1.0 / N_TRUE)
    hid = jnp.maximum(u @ wd1_ref[...] + bd1_ref[...], 0.0)
    out_ref[...] = hid @ wd2_ref[...] + bd2_ref[...]


def _tc_decode(x_pad, agg2, deg32, W_e, b_e, W_d1, b_d1, W_d2, b_d2):
  out = pl.pallas_call(
      _tc_decode_body,
      grid=(N_BLOCKS,),
      in_specs=[
          pl.BlockSpec((BN_TC, D), lambda i: (i, 0)),
          pl.BlockSpec((NC, BN_TC, D), lambda i: (0, i, 0)),
          pl.BlockSpec((NW, BN_TC), lambda i: (0, i)),
          pl.BlockSpec((D, D), lambda i: (0, 0)),
          pl.BlockSpec((1, D), lambda i: (0, 0)),
          pl.BlockSpec((D, D_HID), lambda i: (0, 0)),
          pl.BlockSpec((1, D_HID), lambda i: (0, 0)),
          pl.BlockSpec((D_HID, D_OUT), lambda i: (0, 0)),
          pl.BlockSpec((1, D_OUT), lambda i: (0, 0)),
      ],
      out_specs=pl.BlockSpec((1, D_OUT), lambda i: (0, 0)),
      out_shape=jax.ShapeDtypeStruct((1, D_OUT), jnp.float32),
      scratch_shapes=[pltpu.VMEM((8, D), jnp.float32)],
  )(x_pad, agg2, deg32, W_e, b_e.reshape(1, D), W_d1, b_d1.reshape(1, D_HID),
    W_d2, b_d2.reshape(1, D_OUT))
  return out.reshape(D_OUT)


@jax.jit
def kernel(x, edge_index, W_e, b_e, W_d1, b_d1, W_d2, b_d2):
  src = edge_index[0].astype(jnp.int32)
  dst = edge_index[1].astype(jnp.int32)
  pad_idx = N_TRUE + jnp.arange(E_PAD - E_TRUE, dtype=jnp.int32) % (
      N_PAD - N_TRUE)
  src_p = jnp.concatenate([src, pad_idx])
  dst_p = jnp.concatenate([dst, pad_idx])
  x_pad = jnp.pad(x, ((0, N_PAD - N_TRUE), (0, 0)))
  zeros2d = jnp.zeros((N_PAD, D), jnp.float32)

  agg2, deg32 = _sc_segment_sum(x_pad, src_p, dst_p, zeros2d)
  return _tc_decode(x_pad, agg2, deg32, W_e, b_e, W_d1, b_d1, W_d2, b_d2)


# speedup vs baseline: 2.9377x; 1.1201x over previous
import functools

import jax
import jax.numpy as jnp
from jax import lax
from jax.experimental import pallas as pl
from jax.experimental.pallas import tpu as pltpu
from jax.experimental.pallas import tpu_sc as plsc

N_TRUE = 10000
E_TRUE = 320000
D = 128
D_HID = 256
D_OUT = 64

NC = 2
NS = 16
NW = NC * NS

CHUNK = 128
N_PAD = 10240
ROWS_PER_SUB = N_PAD // NS
K0 = 80
K1 = 78
TOTAL_CHUNKS = NS * (K0 + K1)
E_PAD = TOTAL_CHUNKS * CHUNK

BN_TC = 1024
N_BLOCKS = N_PAD // BN_TC


def _sc_segment_sum(table, src_p, dst_p, zeros2d):
  mesh = plsc.VectorSubcoreMesh(core_axis_name="c", subcore_axis_name="s")

  @functools.partial(
      pl.kernel,
      mesh=mesh,
      out_type=[
          jax.ShapeDtypeStruct((NC, N_PAD, D), jnp.float32),
          jax.ShapeDtypeStruct((NW, N_PAD), jnp.float32),
      ],
      compiler_params=pltpu.CompilerParams(needs_layout_passes=False),
      scratch_types=[
          [pltpu.VMEM((CHUNK,), jnp.int32)] * 2,
          [pltpu.VMEM((CHUNK,), jnp.int32)] * 2,
          [pltpu.VMEM((CHUNK, D), jnp.float32)] * 2,
          pltpu.VMEM((N_PAD,), jnp.float32),
          pltpu.VMEM_SHARED((N_PAD, D), jnp.float32),
          [pltpu.SemaphoreType.DMA] * 2,
          [pltpu.SemaphoreType.DMA] * 2,
      ],
  )
  def seg_kernel(tab_hbm, src_hbm, dst_hbm, z_hbm, agg_out, deg_out,
                 src_v, dst_v, rows, deg_v, agg_sh, semg, semi):
    c = lax.axis_index("c")
    s = lax.axis_index("s")
    wid = c * NS + s

    pltpu.sync_copy(z_hbm.at[pl.ds(s * ROWS_PER_SUB, ROWS_PER_SUB)],
                    agg_sh.at[pl.ds(s * ROWS_PER_SUB, ROWS_PER_SUB)])

    zeros16 = jnp.zeros((16,), jnp.float32)
    def _zero_deg(i, carry):
      deg_v[pl.ds(i * 16, 16)] = zeros16
      return carry
    lax.fori_loop(0, N_PAD // 16, _zero_deg, 0)

    plsc.subcore_barrier()

    ones16 = jnp.full((16,), 1.0, jnp.float32)
    nk = jnp.where(c == 0, K0, K1)
    base_chunk = jnp.where(c == 0, s * K0, NS * K0 + s * K1)
    base = base_chunk * CHUNK
    last_off = base + (nk - 1) * CHUNK

    for b in range(2):
      off = base + b * CHUNK
      pltpu.async_copy(src_hbm.at[pl.ds(off, CHUNK)], src_v[b], semi[b])
      pltpu.async_copy(dst_hbm.at[pl.ds(off, CHUNK)], dst_v[b], semi[b])

    def _edge_round(i, carry):
      k = 2 * i
      off0 = base + k * CHUNK
      off1 = base + (k + 1) * CHUNK
      pltpu.make_async_copy(src_hbm.at[pl.ds(off0, CHUNK)], src_v[0],
                            semi[0]).wait()
      pltpu.make_async_copy(dst_hbm.at[pl.ds(off0, CHUNK)], dst_v[0],
                            semi[0]).wait()
      pltpu.async_copy(tab_hbm.at[src_v[0]], rows[0], semg[0])
      for j in range(CHUNK // 16):
        idx = dst_v[0][pl.ds(j * 16, 16)]
        plsc.addupdate_scatter(deg_v, [idx], ones16)
      pltpu.make_async_copy(src_hbm.at[pl.ds(off1, CHUNK)], src_v[1],
                            semi[1]).wait()
      pltpu.make_async_copy(dst_hbm.at[pl.ds(off1, CHUNK)], dst_v[1],
                            semi[1]).wait()
      pltpu.make_async_copy(tab_hbm.at[src_v[0]], rows[0], semg[0]).wait()
      pltpu.async_copy(tab_hbm.at[src_v[1]], rows[1], semg[1])
      pltpu.sync_copy(rows[0], agg_sh.at[dst_v[0]], add=True)
      offn0 = base + jnp.minimum(k + 2, nk - 1) * CHUNK
      pltpu.async_copy(src_hbm.at[pl.ds(offn0, CHUNK)], src_v[0], semi[0])
      pltpu.async_copy(dst_hbm.at[pl.ds(offn0, CHUNK)], dst_v[0], semi[0])
      for j in range(CHUNK // 16):
        idx = dst_v[1][pl.ds(j * 16, 16)]
        plsc.addupdate_scatter(deg_v, [idx], ones16)
      pltpu.make_async_copy(tab_hbm.at[src_v[1]], rows[1], semg[1]).wait()
      pltpu.sync_copy(rows[1], agg_sh.at[dst_v[1]], add=True)
      offn1 = base + jnp.minimum(k + 3, nk - 1) * CHUNK
      pltpu.async_copy(src_hbm.at[pl.ds(offn1, CHUNK)], src_v[1], semi[1])
      pltpu.async_copy(dst_hbm.at[pl.ds(offn1, CHUNK)], dst_v[1], semi[1])
      return carry

    lax.fori_loop(0, nk // 2, _edge_round, 0)

    for b in range(2):
      pltpu.make_async_copy(src_hbm.at[pl.ds(last_off, CHUNK)], src_v[b],
                            semi[b]).wait()
      pltpu.make_async_copy(dst_hbm.at[pl.ds(last_off, CHUNK)], dst_v[b],
                            semi[b]).wait()

    plsc.subcore_barrier()

    pltpu.sync_copy(agg_sh.at[pl.ds(s * ROWS_PER_SUB, ROWS_PER_SUB)],
                    agg_out.at[c, pl.ds(s * ROWS_PER_SUB, ROWS_PER_SUB)])
    pltpu.sync_copy(deg_v, deg_out.at[wid])

  return seg_kernel(table, src_p, dst_p, zeros2d)


def _tc_decode_body(x_ref, agg_ref, deg_ref, we_ref, be_ref, wd1_ref, bd1_ref,
                    wd2_ref, bd2_ref, out_ref, u_acc):
  i = pl.program_id(0)

  @pl.when(i == 0)
  def _():
    u_acc[...] = jnp.zeros((8, D), jnp.float32)

  agg = agg_ref[0] + agg_ref[1]
  deg = jnp.sum(deg_ref[...], axis=0)[:, None]
  mean = agg / jnp.maximum(deg, 1.0)
  z = (x_ref[...] + mean) @ we_ref[...] + be_ref[...]
  h = jnp.maximum(z, 0.0)

  row = i * BN_TC + lax.broadcasted_iota(jnp.int32, (BN_TC, 1), 0)
  h = jnp.where(row < N_TRUE, h, 0.0)
  u_acc[0:1, :] += jnp.sum(h, axis=0, keepdims=True)

  @pl.when(i == N_BLOCKS - 1)
  def _():
    u = u_acc[0:1, :] * (1.0 / N_TRUE)
    hid = jnp.maximum(u @ wd1_ref[...] + bd1_ref[...], 0.0)
    out_ref[...] = hid @ wd2_ref[...] + bd2_ref[...]


def _tc_decode(x_pad, agg2, deg32, W_e, b_e, W_d1, b_d1, W_d2, b_d2):
  out = pl.pallas_call(
      _tc_decode_body,
      grid=(N_BLOCKS,),
      in_specs=[
          pl.BlockSpec((BN_TC, D), lambda i: (i, 0)),
          pl.BlockSpec((NC, BN_TC, D), lambda i: (0, i, 0)),
          pl.BlockSpec((NW, BN_TC), lambda i: (0, i)),
          pl.BlockSpec((D, D), lambda i: (0, 0)),
          pl.BlockSpec((1, D), lambda i: (0, 0)),
          pl.BlockSpec((D, D_HID), lambda i: (0, 0)),
          pl.BlockSpec((1, D_HID), lambda i: (0, 0)),
          pl.BlockSpec((D_HID, D_OUT), lambda i: (0, 0)),
          pl.BlockSpec((1, D_OUT), lambda i: (0, 0)),
      ],
      out_specs=pl.BlockSpec((1, D_OUT), lambda i: (0, 0)),
      out_shape=jax.ShapeDtypeStruct((1, D_OUT), jnp.float32),
      scratch_shapes=[pltpu.VMEM((8, D), jnp.float32)],
  )(x_pad, agg2, deg32, W_e, b_e.reshape(1, D), W_d1, b_d1.reshape(1, D_HID),
    W_d2, b_d2.reshape(1, D_OUT))
  return out.reshape(D_OUT)


@jax.jit
def kernel(x, edge_index, W_e, b_e, W_d1, b_d1, W_d2, b_d2):
  src = edge_index[0].astype(jnp.int32)
  dst = edge_index[1].astype(jnp.int32)
  pad_idx = N_TRUE + jnp.arange(E_PAD - E_TRUE, dtype=jnp.int32) % (
      N_PAD - N_TRUE)
  src_p = jnp.concatenate([src, pad_idx])
  dst_p = jnp.concatenate([dst, pad_idx])
  x_pad = jnp.pad(x, ((0, N_PAD - N_TRUE), (0, 0)))
  zeros2d = jnp.zeros((N_PAD, D), jnp.float32)

  agg2, deg32 = _sc_segment_sum(x_pad, src_p, dst_p, zeros2d)
  return _tc_decode(x_pad, agg2, deg32, W_e, b_e, W_d1, b_d1, W_d2, b_d2)
